# baseline (speedup 1.0000x reference)
import jax
import jax.numpy as jnp
from jax import lax
from jax.experimental import pallas as pl
from jax.experimental.pallas import tpu as pltpu
from jax.experimental.pallas import tpu_sc as plsc

_B = 16384
_D = 32
_V = 1_000_000
_NC = 2
_NS = 16
_NW = _NC * _NS
_BPW = _B // _NW


def _sc_gather_body(uidx_hbm, iidx_hbm, utf, itf, ue2, ie2,
                    uidx_v, iidx_v, eidx_v, urows_v, irows_v, sem_u, sem_i):
    wid = lax.axis_index("s") * _NC + lax.axis_index("c")
    base = wid * _BPW
    pltpu.sync_copy(uidx_hbm.at[pl.ds(base, _BPW)], uidx_v)
    pltpu.sync_copy(iidx_hbm.at[pl.ds(base, _BPW)], iidx_v)

    def chunk(j):
        ru = uidx_v[pl.ds(j * 16, 16)]
        ri = iidx_v[pl.ds(j * 16, 16)]
        for d in range(_D):
            eidx_v[d, pl.ds(j * 16, 16)] = ru + d * _V
            eidx_v[_D + d, pl.ds(j * 16, 16)] = ri + d * _V

    pl.loop(0, _BPW // 16)(chunk)

    copies = []
    for d in range(_D):
        copies.append(pltpu.async_copy(
            utf.at[eidx_v.at[d]], urows_v.at[d], sem_u))
        copies.append(pltpu.async_copy(
            itf.at[eidx_v.at[_D + d]], irows_v.at[d], sem_i))
    for c in copies:
        c.wait()

    pltpu.sync_copy(urows_v, ue2.at[:, pl.ds(base, _BPW)])
    pltpu.sync_copy(irows_v, ie2.at[:, pl.ds(base, _BPW)])


def _build_sc_gather():
    return pl.kernel(
        _sc_gather_body,
        out_type=(jax.ShapeDtypeStruct((_D, _B), jnp.float32),
                  jax.ShapeDtypeStruct((_D, _B), jnp.float32)),
        mesh=plsc.VectorSubcoreMesh(core_axis_name="c", subcore_axis_name="s",
                                    num_cores=_NC, num_subcores=_NS),
        scratch_types=[
            pltpu.VMEM((_BPW,), jnp.int32),
            pltpu.VMEM((_BPW,), jnp.int32),
            pltpu.VMEM((2 * _D, _BPW), jnp.int32),
            pltpu.VMEM((_D, _BPW), jnp.float32),
            pltpu.VMEM((_D, _BPW), jnp.float32),
            pltpu.SemaphoreType.DMA,
            pltpu.SemaphoreType.DMA,
        ],
        compiler_params=pltpu.CompilerParams(use_tc_tiling_on_sc=False),
    )


_BLK = 2048


def _mlp_body(ue_ref, ie_ref, w1u_ref, w1i_ref, b1_ref, w2_ref, out_ref):
    dn = (((0,), (0,)), ((), ()))
    h = lax.dot_general(w1u_ref[...], ue_ref[...], dn,
                        preferred_element_type=jnp.float32)
    h = h + lax.dot_general(w1i_ref[...], ie_ref[...], dn,
                            preferred_element_type=jnp.float32)
    h = jnp.maximum(h + b1_ref[...], 0.0)
    out_ref[...] = lax.dot_general(w2_ref[...], h, dn,
                                   preferred_element_type=jnp.float32)


def _build_mlp():
    return pl.pallas_call(
        _mlp_body,
        grid=(_B // _BLK,),
        in_specs=[
            pl.BlockSpec((_D, _BLK), lambda i: (0, i)),
            pl.BlockSpec((_D, _BLK), lambda i: (0, i)),
            pl.BlockSpec((_D, _D), lambda i: (0, 0)),
            pl.BlockSpec((_D, _D), lambda i: (0, 0)),
            pl.BlockSpec((_D, 1), lambda i: (0, 0)),
            pl.BlockSpec((_D, 1), lambda i: (0, 0)),
        ],
        out_specs=pl.BlockSpec((1, _BLK), lambda i: (0, i)),
        out_shape=jax.ShapeDtypeStruct((1, _B), jnp.float32),
    )


def kernel(x, user_table, item_table, W1, b1, W2):
    uidx = x[:, 0].astype(jnp.int32)
    iidx = x[:, 1].astype(jnp.int32)
    utf = user_table.T.reshape(_D * _V)
    itf = item_table.T.reshape(_D * _V)
    ue2, ie2 = _build_sc_gather()(uidx, iidx, utf, itf)
    out_t = _build_mlp()(ue2, ie2, W1[:_D], W1[_D:], b1.reshape(_D, 1), W2)
    return (out_t.T, ue2.T, ie2.T)
